# baseline (speedup 1.0000x reference)
import functools

import jax
import jax.numpy as jnp
from jax import lax
from jax.experimental import pallas as pl
from jax.experimental.pallas import tpu as pltpu
from jax.experimental.pallas import tpu_sc as plsc

D = 1024
B = 8192
NC = 2
NS = 16
NW = NC * NS
B_PER_W = B // NW
C = 16
NCH = B_PER_W // C


NB = 7


def _make_sc_gather():
    mesh = plsc.VectorSubcoreMesh(core_axis_name="c", subcore_axis_name="s")

    @functools.partial(
        pl.kernel,
        mesh=mesh,
        out_type=jax.ShapeDtypeStruct((B, D), jnp.float32),
        scratch_types=(
            [pltpu.VMEM((B_PER_W,), jnp.int32)]
            + [pltpu.VMEM_SHARED((NS, B_PER_W), jnp.int32)]
            + [pltpu.SMEM((B_PER_W,), jnp.int32)]
            + [pltpu.SemaphoreType.DMA]
        ),
    )
    def gather_kernel(table_hbm, idx_hbm, out_hbm, idx_v, idx_sh, idx_s, sem):
        sid = lax.axis_index("s")
        wid = sid * NC + lax.axis_index("c")
        base = wid * B_PER_W
        pltpu.sync_copy(idx_hbm.at[pl.ds(base, B_PER_W)], idx_v)
        pltpu.sync_copy(idx_v, idx_sh.at[sid])
        pltpu.sync_copy(idx_sh.at[sid], idx_s)

        cps = [pltpu.async_copy(
                   table_hbm.at[pl.ds(idx_s[i], 1)],
                   out_hbm.at[pl.ds(base + i, 1)],
                   sem)
               for i in range(B_PER_W)]
        for cp in cps:
            cp.wait()

    return gather_kernel


_sc_gather = _make_sc_gather()


@jax.jit
def kernel(token_ids, base_embeddings):
    tokens = token_ids.reshape(-1).astype(jnp.int32)
    out = _sc_gather(base_embeddings, tokens)
    return out.reshape(token_ids.shape + (base_embeddings.shape[-1],))


# speedup vs baseline: 23.8673x; 23.8673x over previous
import functools

import jax
import jax.numpy as jnp
from jax import lax
from jax.experimental import pallas as pl
from jax.experimental.pallas import tpu as pltpu
from jax.experimental.pallas import tpu_sc as plsc

D = 1024
B = 8192
NC = 2
NS = 16
NW = NC * NS
B_PER_W = B // NW
C = 16
NB = 6


def _make_sc_gather():
    mesh = plsc.VectorSubcoreMesh(core_axis_name="c", subcore_axis_name="s")

    @functools.partial(
        pl.kernel,
        mesh=mesh,
        out_type=jax.ShapeDtypeStruct((B, D), jnp.float32),
        scratch_types=(
            [pltpu.VMEM((B_PER_W,), jnp.int32)]
            + [pltpu.VMEM((NB * C, D), jnp.float32)]
            + [pltpu.SemaphoreType.DMA for _ in range(2 * NB)]
        ),
    )
    def gather_kernel(table_hbm, idx_hbm, out_hbm, idx_v, pool, *sems):
        gsems = sems[:NB]
        wsems = sems[NB:2 * NB]

        wid = lax.axis_index("s") * NC + lax.axis_index("c")
        base = wid * B_PER_W
        pltpu.sync_copy(idx_hbm.at[pl.ds(base, B_PER_W)], idx_v)

        def pipeline(c_rows, nb):
            nch = B_PER_W // c_rows

            def start_gather(c, b):
                return pltpu.async_copy(
                    table_hbm.at[idx_v.at[pl.ds(c * c_rows, c_rows)]],
                    pool.at[pl.ds(b * c_rows, c_rows)], gsems[b])

            def start_write(c, b):
                return pltpu.async_copy(
                    pool.at[pl.ds(b * c_rows, c_rows)],
                    out_hbm.at[pl.ds(base + c * c_rows, c_rows)], wsems[b])

            g = [start_gather(c, c % nb) for c in range(min(nb, nch))]
            w = [None] * nb
            for c in range(nch):
                b = c % nb
                g[b].wait()
                w[b] = start_write(c, b)
                if c + nb < nch:
                    w[b].wait()
                    g[b] = start_gather(c + nb, b)
            for c in range(max(0, nch - nb), nch):
                w[c % nb].wait()

        @pl.when(wid % 2 == 0)
        def _():
            pipeline(C, NB)

        @pl.when(wid % 2 != 0)
        def _():
            pipeline(2 * C, NB // 2)

    return gather_kernel


_sc_gather = _make_sc_gather()


@jax.jit
def kernel(token_ids, base_embeddings):
    tokens = token_ids.reshape(-1).astype(jnp.int32)
    out = _sc_gather(base_embeddings, tokens)
    return out.reshape(token_ids.shape + (base_embeddings.shape[-1],))
